# baseline (speedup 1.0000x reference)
import functools

import jax
import jax.numpy as jnp
from jax import lax
from jax.experimental import pallas as pl
from jax.experimental.pallas import tpu as pltpu
from jax.experimental.pallas import tpu_sc as plsc

_LANES = 16
_NBUF = 4


def _make_sc_add(B, S, D, NC, NS, CH, UNROLL):
    NW = NC * NS
    rows_per_w = S // NW
    n_chunks = rows_per_w // CH
    elems = CH * D
    steps = elems // (UNROLL * _LANES)
    n_tiles = n_chunks * B
    mesh = plsc.VectorSubcoreMesh(core_axis_name="c", subcore_axis_name="s")

    @functools.partial(
        pl.kernel,
        out_type=jax.ShapeDtypeStruct((B * S * D,), jnp.float32),
        mesh=mesh,
        scratch_types=[
            pltpu.VMEM((elems,), jnp.float32),
            pltpu.VMEM((_NBUF, elems), jnp.float32),
            pltpu.SemaphoreType.DMA((_NBUF,)),
            pltpu.SemaphoreType.DMA((_NBUF,)),
        ],
    )
    def sc_add(x_hbm, pos_hbm, out_hbm, posb, xbuf, lsem, ssem):
        wid = lax.axis_index("s") * NC + lax.axis_index("c")
        sbase = wid * rows_per_w

        def xoff(t):
            c, b = divmod(t, B)
            return (b * S + sbase + c * CH) * D

        def start_load(t):
            return pltpu.async_copy(
                x_hbm.at[pl.ds(xoff(t), elems)],
                xbuf.at[t % _NBUF], lsem.at[t % _NBUF])

        def start_store(t):
            return pltpu.async_copy(
                xbuf.at[t % _NBUF],
                out_hbm.at[pl.ds(xoff(t), elems)], ssem.at[t % _NBUF])

        loads = {t: start_load(t) for t in range(min(_NBUF, n_tiles))}
        stores = {}
        unretired = set()

        for c in range(n_chunks):
            pltpu.sync_copy(
                pos_hbm.at[pl.ds((sbase + c * CH) * D, elems)], posb)
            for b in range(B):
                t = c * B + b
                pt, nt = t - 2, t - 2 + _NBUF
                if pt >= 0 and nt < n_tiles:
                    stores[pt].wait()
                    unretired.discard(pt)
                    loads[nt] = start_load(nt)
                k = t % _NBUF
                loads[t].wait()

                def addstep(j, carry):
                    base = j * (UNROLL * _LANES)
                    for u in range(UNROLL):
                        o = base + u * _LANES
                        plsc.addupdate(xbuf.at[k, pl.ds(o, _LANES)],
                                       posb[pl.ds(o, _LANES)])
                    return carry

                lax.fori_loop(0, steps, addstep, 0)
                stores[t] = start_store(t)
                unretired.add(t)

        for t in sorted(unretired):
            stores[t].wait()

    return sc_add


def kernel(x, pos_table):
    B, S, D = x.shape
    info = plsc.get_sparse_core_info()
    NC, NS = info.num_cores, info.num_subcores
    out = _make_sc_add(B, S, D, NC, NS, CH=16, UNROLL=16)(
        x.reshape(-1), pos_table[:S].reshape(-1))
    return out.reshape(B, S, D)


# speedup vs baseline: 6.4417x; 6.4417x over previous
import jax
import jax.numpy as jnp
from jax.experimental import pallas as pl


def _add_body(x_ref, pos_ref, o_ref):
    o_ref[...] = x_ref[...] + pos_ref[...]


def kernel(x, pos_table):
    B, S, D = x.shape
    TS = 512
    grid = (S // TS, B)
    return pl.pallas_call(
        _add_body,
        grid=grid,
        in_specs=[
            pl.BlockSpec((1, TS, D), lambda i, j: (j, i, 0)),
            pl.BlockSpec((TS, D), lambda i, j: (i, 0)),
        ],
        out_specs=pl.BlockSpec((1, TS, D), lambda i, j: (j, i, 0)),
        out_shape=jax.ShapeDtypeStruct((B, S, D), x.dtype),
    )(x, pos_table[:S])


# speedup vs baseline: 7.0396x; 1.0928x over previous
import jax
import jax.numpy as jnp
from jax.experimental import pallas as pl


def _add_body(x_ref, pos_ref, o_ref):
    o_ref[...] = x_ref[...] + pos_ref[...]


def kernel(x, pos_table):
    B, S, D = x.shape
    TS = 1024
    grid = (S // TS, B)
    return pl.pallas_call(
        _add_body,
        grid=grid,
        in_specs=[
            pl.BlockSpec((1, TS, D), lambda i, j: (j, i, 0)),
            pl.BlockSpec((TS, D), lambda i, j: (i, 0)),
        ],
        out_specs=pl.BlockSpec((1, TS, D), lambda i, j: (j, i, 0)),
        out_shape=jax.ShapeDtypeStruct((B, S, D), x.dtype),
    )(x, pos_table[:S])


# speedup vs baseline: 7.5948x; 1.0789x over previous
import jax
import jax.numpy as jnp
from jax.experimental import pallas as pl


def _add_body(x_ref, pos_ref, o_ref):
    o_ref[...] = x_ref[...] + pos_ref[...]


def kernel(x, pos_table):
    B, S, D = x.shape
    TS = 2048
    grid = (S // TS, B)
    return pl.pallas_call(
        _add_body,
        grid=grid,
        in_specs=[
            pl.BlockSpec((1, TS, D), lambda i, j: (j, i, 0)),
            pl.BlockSpec((TS, D), lambda i, j: (i, 0)),
        ],
        out_specs=pl.BlockSpec((1, TS, D), lambda i, j: (j, i, 0)),
        out_shape=jax.ShapeDtypeStruct((B, S, D), x.dtype),
    )(x, pos_table[:S])
